# baseline (speedup 1.0000x reference)
import jax
import jax.numpy as jnp
from jax import lax
from jax.experimental import pallas as pl
from jax.experimental.pallas import tpu as pltpu
from jax.experimental.pallas import tpu_sc as plsc

TOP_K = 2
TS = 512
_NEG = -3.0e38


def _sc_routing_body(scores_hbm, idx_hbm, w_hbm, scores_v, idx_v, w_v):
    cid = lax.axis_index("c")
    sid = lax.axis_index("s")

    @pl.when(jnp.logical_and(cid == 0, sid == 0))
    def _():
        pltpu.sync_copy(scores_hbm, scores_v)
        lanes = lax.iota(jnp.int32, 16)
        valid = lanes < 8
        idx_acc = jnp.zeros((16,), jnp.int32)
        w_acc = jnp.zeros((16,), jnp.float32)
        for b in range(4):
            row = scores_v[pl.ds(b * 16, 16)]
            v = jnp.where(valid, row, _NEG)
            sk, sv = plsc.sort_key_val(v, lanes, descending=True)
            m1 = sk[0]
            m2 = sk[1]
            i0 = sv[0]
            i1 = sv[1]
            ev = jnp.exp(jnp.full((16,), m2 - m1, jnp.float32))
            w0v = 1.0 / (1.0 + ev)
            w0 = w0v[0]
            w1 = 1.0 - w0
            idx_acc = jnp.where(lanes == 2 * b, i0, idx_acc)
            idx_acc = jnp.where(lanes == 2 * b + 1, i1, idx_acc)
            w_acc = jnp.where(lanes == 2 * b, w0, w_acc)
            w_acc = jnp.where(lanes == 2 * b + 1, w1, w_acc)
        idx_v[...] = idx_acc
        w_v[...] = w_acc
        pltpu.sync_copy(idx_v, idx_hbm)
        pltpu.sync_copy(w_v, w_hbm)


def _sc_routing(scores):
    B, P = scores.shape
    pad = jnp.full((B, 16 - P), _NEG, scores.dtype)
    flat = jnp.concatenate([scores, pad], axis=1).reshape(-1)
    mesh = plsc.VectorSubcoreMesh(core_axis_name="c", subcore_axis_name="s")
    idx16, w16 = pl.kernel(
        _sc_routing_body,
        out_type=(
            jax.ShapeDtypeStruct((16,), jnp.int32),
            jax.ShapeDtypeStruct((16,), jnp.float32),
        ),
        mesh=mesh,
        compiler_params=pltpu.CompilerParams(needs_layout_passes=False),
        scratch_types=[
            pltpu.VMEM((64,), jnp.float32),
            pltpu.VMEM((16,), jnp.int32),
            pltpu.VMEM((16,), jnp.float32),
        ],
    )(flat)
    top_idx = idx16[: 2 * B].reshape(B, 2)
    w = w16[: 2 * B].reshape(B, 2)
    return top_idx, w


def _fold_kernel(idx_ref, w_ref, p0_ref, p1_ref, reexp_ref, out_ref):
    b = pl.program_id(0)
    w0 = w_ref[b, 0]
    w1 = w_ref[b, 1]
    merged = p0_ref[0, 0] * w0 + p1_ref[0, 0] * w1
    for p in range(out_ref.shape[1]):
        out_ref[0, p] = merged + reexp_ref[p]


def kernel(probes, scores, re_expand):
    B, P, S, D = probes.shape
    top_idx, w = _sc_routing(scores)

    grid = (B, S // TS)

    def probe_spec(k):
        def imap(b, s, idx_ref, w_ref):
            return (b, idx_ref[b, k], s, 0)
        return pl.BlockSpec((1, 1, TS, D), imap)

    out_spec = pl.BlockSpec((1, P, TS, D), lambda b, s, idx_ref, w_ref: (b, 0, s, 0))
    reexp_spec = pl.BlockSpec((P, D), lambda b, s, idx_ref, w_ref: (0, 0))

    grid_spec = pltpu.PrefetchScalarGridSpec(
        num_scalar_prefetch=2,
        grid=grid,
        in_specs=[probe_spec(0), probe_spec(1), reexp_spec],
        out_specs=out_spec,
    )

    return pl.pallas_call(
        _fold_kernel,
        grid_spec=grid_spec,
        out_shape=jax.ShapeDtypeStruct((B, P, S, D), probes.dtype),
        compiler_params=pltpu.CompilerParams(
            dimension_semantics=("parallel", "arbitrary"),
        ),
    )(top_idx, w, probes, probes, re_expand)


# speedup vs baseline: 1.0220x; 1.0220x over previous
import jax
import jax.numpy as jnp
from jax import lax
from jax.experimental import pallas as pl
from jax.experimental.pallas import tpu as pltpu
from jax.experimental.pallas import tpu_sc as plsc

TOP_K = 2
TS = 512
_NEG = -3.0e38


def _sc_routing_body(scores_hbm, idx_hbm, w_hbm, scores_v, idx_v, w_v):
    cid = lax.axis_index("c")
    sid = lax.axis_index("s")

    @pl.when(jnp.logical_and(cid == 0, sid == 0))
    def _():
        pltpu.sync_copy(scores_hbm, scores_v)
        lanes = lax.iota(jnp.int32, 16)
        idx_acc = jnp.zeros((16,), jnp.int32)
        w_acc = jnp.zeros((16,), jnp.float32)
        for half in range(2):
            pair = scores_v[pl.ds(half * 16, 16)]
            for sub in range(2):
                b = 2 * half + sub
                in_row = (lanes >= 8 * sub) & (lanes < 8 * (sub + 1))
                v = jnp.where(in_row, pair, _NEG)
                sk, sv = plsc.sort_key_val(v, lanes - 8 * sub, descending=True)
                m1 = sk[0]
                m2 = sk[1]
                i0 = sv[0]
                i1 = sv[1]
                ev = jnp.exp(jnp.full((16,), m2 - m1, jnp.float32))
                w0v = 1.0 / (1.0 + ev)
                w0 = w0v[0]
                w1 = 1.0 - w0
                idx_acc = jnp.where(lanes == 2 * b, i0, idx_acc)
                idx_acc = jnp.where(lanes == 2 * b + 1, i1, idx_acc)
                w_acc = jnp.where(lanes == 2 * b, w0, w_acc)
                w_acc = jnp.where(lanes == 2 * b + 1, w1, w_acc)
        idx_v[...] = idx_acc
        w_v[...] = w_acc
        pltpu.sync_copy(idx_v, idx_hbm)
        pltpu.sync_copy(w_v, w_hbm)


def _sc_routing(scores_flat):
    mesh = plsc.VectorSubcoreMesh(core_axis_name="c", subcore_axis_name="s")
    return pl.kernel(
        _sc_routing_body,
        out_type=(
            jax.ShapeDtypeStruct((16,), jnp.int32),
            jax.ShapeDtypeStruct((16,), jnp.float32),
        ),
        mesh=mesh,
        compiler_params=pltpu.CompilerParams(needs_layout_passes=False),
        scratch_types=[
            pltpu.VMEM((32,), jnp.float32),
            pltpu.VMEM((16,), jnp.int32),
            pltpu.VMEM((16,), jnp.float32),
        ],
    )(scores_flat)


def _fold_kernel(idx_ref, w_ref, p0_ref, p1_ref, reexp_ref, out_ref):
    b = pl.program_id(0)
    w0 = w_ref[2 * b]
    w1 = w_ref[2 * b + 1]
    merged = p0_ref[0, 0] * w0 + p1_ref[0, 0] * w1
    for p in range(out_ref.shape[1]):
        out_ref[0, p] = merged + reexp_ref[p]


def kernel(probes, scores, re_expand):
    B, P, S, D = probes.shape
    idx16, w16 = _sc_routing(scores.reshape(-1))

    grid = (B, S // TS)

    def probe_spec(k):
        def imap(b, s, idx_ref, w_ref):
            return (b, idx_ref[2 * b + k], s, 0)
        return pl.BlockSpec((1, 1, TS, D), imap)

    out_spec = pl.BlockSpec((1, P, TS, D), lambda b, s, idx_ref, w_ref: (b, 0, s, 0))
    reexp_spec = pl.BlockSpec((P, D), lambda b, s, idx_ref, w_ref: (0, 0))

    grid_spec = pltpu.PrefetchScalarGridSpec(
        num_scalar_prefetch=2,
        grid=grid,
        in_specs=[probe_spec(0), probe_spec(1), reexp_spec],
        out_specs=out_spec,
    )

    return pl.pallas_call(
        _fold_kernel,
        grid_spec=grid_spec,
        out_shape=jax.ShapeDtypeStruct((B, P, S, D), probes.dtype),
        compiler_params=pltpu.CompilerParams(
            dimension_semantics=("parallel", "arbitrary"),
        ),
    )(idx16, w16, probes, probes, re_expand)


# speedup vs baseline: 1.0373x; 1.0149x over previous
import jax
import jax.numpy as jnp
from jax import lax
from jax.experimental import pallas as pl
from jax.experimental.pallas import tpu as pltpu
from jax.experimental.pallas import tpu_sc as plsc

TOP_K = 2
TS = 512
_NEG = -3.0e38


def _sc_routing_body(scores_hbm, idx_hbm, w_hbm, scores_v, idx_v, w_v):
    cid = lax.axis_index("c")
    sid = lax.axis_index("s")

    @pl.when(jnp.logical_and(cid == 0, sid == 0))
    def _():
        pltpu.sync_copy(scores_hbm, scores_v)
        lanes = lax.iota(jnp.int32, 16)
        idx_acc = jnp.zeros((16,), jnp.int32)
        w_acc = jnp.zeros((16,), jnp.float32)
        for half in range(2):
            pair = scores_v[pl.ds(half * 16, 16)]
            for sub in range(2):
                b = 2 * half + sub
                in_row = (lanes >= 8 * sub) & (lanes < 8 * (sub + 1))
                v = jnp.where(in_row, pair, _NEG)
                sk, sv = plsc.sort_key_val(v, lanes - 8 * sub, descending=True)
                m1 = sk[0]
                m2 = sk[1]
                i0 = sv[0]
                i1 = sv[1]
                ev = jnp.exp(jnp.full((16,), m2 - m1, jnp.float32))
                w0v = 1.0 / (1.0 + ev)
                w0 = w0v[0]
                w1 = 1.0 - w0
                idx_acc = jnp.where(lanes == 2 * b, i0, idx_acc)
                idx_acc = jnp.where(lanes == 2 * b + 1, i1, idx_acc)
                w_acc = jnp.where(lanes == 2 * b, w0, w_acc)
                w_acc = jnp.where(lanes == 2 * b + 1, w1, w_acc)
        idx_v[...] = idx_acc
        w_v[...] = w_acc
        pltpu.sync_copy(idx_v, idx_hbm)
        pltpu.sync_copy(w_v, w_hbm)


def _sc_routing(scores_flat):
    mesh = plsc.VectorSubcoreMesh(
        core_axis_name="c", subcore_axis_name="s", num_cores=1, num_subcores=1
    )
    return pl.kernel(
        _sc_routing_body,
        out_type=(
            jax.ShapeDtypeStruct((16,), jnp.int32),
            jax.ShapeDtypeStruct((16,), jnp.float32),
        ),
        mesh=mesh,
        compiler_params=pltpu.CompilerParams(needs_layout_passes=False),
        scratch_types=[
            pltpu.VMEM((32,), jnp.float32),
            pltpu.VMEM((16,), jnp.int32),
            pltpu.VMEM((16,), jnp.float32),
        ],
    )(scores_flat)


def _fold_kernel(idx_ref, w_ref, p0_ref, p1_ref, reexp_ref, out_ref):
    b = pl.program_id(0)
    w0 = w_ref[2 * b]
    w1 = w_ref[2 * b + 1]
    merged = p0_ref[0, 0] * w0 + p1_ref[0, 0] * w1
    for p in range(out_ref.shape[1]):
        out_ref[0, p] = merged + reexp_ref[p]


def kernel(probes, scores, re_expand):
    B, P, S, D = probes.shape
    idx16, w16 = _sc_routing(scores.reshape(-1))

    grid = (B, S // TS)

    def probe_spec(k):
        def imap(b, s, idx_ref, w_ref):
            return (b, idx_ref[2 * b + k], s, 0)
        return pl.BlockSpec((1, 1, TS, D), imap)

    out_spec = pl.BlockSpec((1, P, TS, D), lambda b, s, idx_ref, w_ref: (b, 0, s, 0))
    reexp_spec = pl.BlockSpec((P, D), lambda b, s, idx_ref, w_ref: (0, 0))

    grid_spec = pltpu.PrefetchScalarGridSpec(
        num_scalar_prefetch=2,
        grid=grid,
        in_specs=[probe_spec(0), probe_spec(1), reexp_spec],
        out_specs=out_spec,
    )

    return pl.pallas_call(
        _fold_kernel,
        grid_spec=grid_spec,
        out_shape=jax.ShapeDtypeStruct((B, P, S, D), probes.dtype),
        compiler_params=pltpu.CompilerParams(
            dimension_semantics=("parallel", "arbitrary"),
        ),
    )(idx16, w16, probes, probes, re_expand)
